# Initial kernel scaffold; baseline (speedup 1.0000x reference)
import jax
import jax.numpy as jnp
from jax.experimental import pallas as pl


def kernel(x, edge_index, edge_attr, batch, Win, b_in, msg_W1, msg_b1, msg_g1, msg_be1, msg_W2, msg_b2, msg_g2, msg_be2, upd_W1, upd_b1, upd_g1, upd_be1, upd_W2, upd_b2, upd_g2, upd_be2):
    raise NotImplementedError("write your pallas kernel here")


# speedup vs baseline: 1.9027x; 1.9027x over previous
import functools

import jax
import jax.numpy as jnp
from jax import lax
from jax.experimental import pallas as pl
from jax.experimental.pallas import tpu as pltpu
from jax.experimental.pallas import tpu_sc as plsc

N = 10000
E = 160000
EMB = 256
EDGE = 16
L = 4
G = 64
EPS = 1e-5

TE = 2000
TN = 2000
CH = 128
NCH = E // CH
NW = 32
G_IT = -(-NCH // NW)
S_IT = -(-NCH // 16)
RPT = 640
RPT_LAST = N - 15 * RPT

_f32 = jnp.float32


def _sds(shape, dtype=_f32):
    return jax.ShapeDtypeStruct(shape, dtype)


def _sc_gather(P, Q, dst, src):
    mesh = plsc.VectorSubcoreMesh(core_axis_name="c", subcore_axis_name="s")

    @functools.partial(
        pl.kernel,
        mesh=mesh,
        out_type=(_sds((E, EMB)), _sds((E, EMB))),
        scratch_types=[
            pltpu.VMEM((CH,), jnp.int32),
            pltpu.VMEM((CH,), jnp.int32),
            pltpu.VMEM((CH, EMB), _f32),
            pltpu.VMEM((CH, EMB), _f32),
            pltpu.SemaphoreType.DMA,
            pltpu.SemaphoreType.DMA,
        ],
    )
    def k(p_hbm, q_hbm, dst_hbm, src_hbm, pg_hbm, qg_hbm, dv, sv, bp, bq, m1, m2):
        c = lax.axis_index("c")
        s = lax.axis_index("s")
        wid = s * 2 + c

        def body(i, carry):
            cid = i * NW + wid

            @pl.when(cid < NCH)
            def _():
                off = pl.multiple_of(cid * CH, CH)
                pltpu.sync_copy(dst_hbm.at[pl.ds(off, CH)], dv)
                pltpu.sync_copy(src_hbm.at[pl.ds(off, CH)], sv)
                cp = pltpu.async_copy(p_hbm.at[dv], bp, m1)
                cq = pltpu.async_copy(q_hbm.at[sv], bq, m2)
                cp.wait()
                cq.wait()
                pltpu.sync_copy(bp, pg_hbm.at[pl.ds(off, CH)])
                pltpu.sync_copy(bq, qg_hbm.at[pl.ds(off, CH)])

            return carry

        lax.fori_loop(0, G_IT, body, 0)

    return k(P, Q, dst, src)


def _sc_scatter_add(mlo, mhi, dst, zeros_half):
    mesh = plsc.VectorSubcoreMesh(core_axis_name="c", subcore_axis_name="s")
    H = EMB // 2

    @functools.partial(
        pl.kernel,
        mesh=mesh,
        out_type=(_sds((N, H)), _sds((N, H))),
        scratch_types=[
            pltpu.VMEM((CH,), jnp.int32),
            pltpu.VMEM((CH, H), _f32),
            pltpu.VMEM_SHARED((N, H), _f32),
        ],
    )
    def k(mlo_hbm, mhi_hbm, dst_hbm, z_hbm, alo_hbm, ahi_hbm, iv, bm, acc):
        c = lax.axis_index("c")
        s = lax.axis_index("s")

        def rows_copy(get_src, get_dst):
            @pl.when(s < 15)
            def _():
                off = pl.multiple_of(s * RPT, RPT)
                pltpu.sync_copy(get_src(off, RPT), get_dst(off, RPT))

            @pl.when(s == 15)
            def _():
                pltpu.sync_copy(get_src(15 * RPT, RPT_LAST),
                                get_dst(15 * RPT, RPT_LAST))

        rows_copy(lambda o, n: z_hbm.at[pl.ds(o, n)],
                  lambda o, n: acc.at[pl.ds(o, n)])
        plsc.subcore_barrier()

        def phase(m_hbm):
            def body(i, carry):
                cid = i * 16 + s

                @pl.when(cid < NCH)
                def _():
                    off = pl.multiple_of(cid * CH, CH)
                    pltpu.sync_copy(dst_hbm.at[pl.ds(off, CH)], iv)
                    pltpu.sync_copy(m_hbm.at[pl.ds(off, CH)], bm)
                    pltpu.sync_copy(bm, acc.at[iv], add=True)

                return carry

            lax.fori_loop(0, S_IT, body, 0)

        @pl.when(c == 0)
        def _():
            phase(mlo_hbm)

        @pl.when(c == 1)
        def _():
            phase(mhi_hbm)

        plsc.subcore_barrier()

        @pl.when(c == 0)
        def _():
            rows_copy(lambda o, n: acc.at[pl.ds(o, n)],
                      lambda o, n: alo_hbm.at[pl.ds(o, n)])

        @pl.when(c == 1)
        def _():
            rows_copy(lambda o, n: acc.at[pl.ds(o, n)],
                      lambda o, n: ahi_hbm.at[pl.ds(o, n)])

    return k(mlo, mhi, dst, zeros_half)


def _acc_stats(i, y, s_ref, ss_ref):
    ps = jnp.sum(y, axis=0, keepdims=True)
    pss = jnp.sum(y * y, axis=0, keepdims=True)

    @pl.when(i == 0)
    def _():
        s_ref[...] = ps
        ss_ref[...] = pss

    @pl.when(i > 0)
    def _():
        s_ref[...] += ps
        ss_ref[...] += pss


def _full(shape):
    return pl.BlockSpec(shape, lambda i: (0, 0))


def _tile(shape):
    return pl.BlockSpec(shape, lambda i: (i, 0))


def _k_init(x, Win, b_in, W1a, W1b):
    def body(x_ref, w_ref, b_ref, wa_ref, wb_ref, h_ref, p_ref, q_ref):
        h = jnp.dot(x_ref[...], w_ref[...], preferred_element_type=_f32) + b_ref[...]
        h_ref[...] = h
        p_ref[...] = jnp.dot(h, wa_ref[...], preferred_element_type=_f32)
        q_ref[...] = jnp.dot(h, wb_ref[...], preferred_element_type=_f32)

    return pl.pallas_call(
        body,
        grid=(N // TN,),
        in_specs=[_tile((TN, EMB)), _full((EMB, EMB)), _full((1, EMB)),
                  _full((EMB, EMB)), _full((EMB, EMB))],
        out_specs=[_tile((TN, EMB))] * 3,
        out_shape=[_sds((N, EMB))] * 3,
    )(x, Win, b_in, W1a, W1b)


def _edge_y1(pg_ref, qg_ref, ea_ref, wc_ref, b_ref):
    return (pg_ref[...] + qg_ref[...] + b_ref[...]
            + jnp.dot(ea_ref[...], wc_ref[...], preferred_element_type=_f32))


def _k_edge_stats(Pg, Qg, ea, W1c, b1):
    def body(pg_ref, qg_ref, ea_ref, wc_ref, b_ref, s_ref, ss_ref):
        i = pl.program_id(0)
        y = _edge_y1(pg_ref, qg_ref, ea_ref, wc_ref, b_ref)
        _acc_stats(i, y, s_ref, ss_ref)

    return pl.pallas_call(
        body,
        grid=(E // TE,),
        in_specs=[_tile((TE, EMB)), _tile((TE, EMB)), _tile((TE, EDGE)),
                  _full((EDGE, EMB)), _full((1, EMB))],
        out_specs=[_full((1, EMB))] * 2,
        out_shape=[_sds((1, EMB))] * 2,
    )(Pg, Qg, ea, W1c, b1)


def _k_edge_mlp(Pg, Qg, ea, W1c, b1, sc1, sh1, W2, b2):
    def body(pg_ref, qg_ref, ea_ref, wc_ref, b_ref, sc_ref, sh_ref,
             w2_ref, b2_ref, y2_ref, s_ref, ss_ref):
        i = pl.program_id(0)
        y = _edge_y1(pg_ref, qg_ref, ea_ref, wc_ref, b_ref)
        z = jnp.maximum(y * sc_ref[...] + sh_ref[...], 0.0)
        y2 = jnp.dot(z, w2_ref[...], preferred_element_type=_f32) + b2_ref[...]
        y2_ref[...] = y2
        _acc_stats(i, y2, s_ref, ss_ref)

    return pl.pallas_call(
        body,
        grid=(E // TE,),
        in_specs=[_tile((TE, EMB)), _tile((TE, EMB)), _tile((TE, EDGE)),
                  _full((EDGE, EMB)), _full((1, EMB)), _full((1, EMB)),
                  _full((1, EMB)), _full((EMB, EMB)), _full((1, EMB))],
        out_specs=[_tile((TE, EMB)), _full((1, EMB)), _full((1, EMB))],
        out_shape=[_sds((E, EMB)), _sds((1, EMB)), _sds((1, EMB))],
    )(Pg, Qg, ea, W1c, b1, sc1, sh1, W2, b2)


def _k_edge_act(y2, sc2, sh2):
    H = EMB // 2

    def body(y_ref, sc_ref, sh_ref, lo_ref, hi_ref):
        m = jnp.maximum(y_ref[...] * sc_ref[...] + sh_ref[...], 0.0)
        lo_ref[...] = m[:, :H]
        hi_ref[...] = m[:, H:]

    return pl.pallas_call(
        body,
        grid=(E // TE,),
        in_specs=[_tile((TE, EMB)), _full((1, EMB)), _full((1, EMB))],
        out_specs=[_tile((TE, H))] * 2,
        out_shape=[_sds((E, H))] * 2,
    )(y2, sc2, sh2)


def _k_node1(h, alo, ahi, U1a, U1bl, U1bh, ub1):
    H = EMB // 2

    def body(h_ref, lo_ref, hi_ref, wa_ref, wl_ref, wh_ref, b_ref,
             u_ref, s_ref, ss_ref):
        i = pl.program_id(0)
        u = (jnp.dot(h_ref[...], wa_ref[...], preferred_element_type=_f32)
             + jnp.dot(lo_ref[...], wl_ref[...], preferred_element_type=_f32)
             + jnp.dot(hi_ref[...], wh_ref[...], preferred_element_type=_f32)
             + b_ref[...])
        u_ref[...] = u
        _acc_stats(i, u, s_ref, ss_ref)

    return pl.pallas_call(
        body,
        grid=(N // TN,),
        in_specs=[_tile((TN, EMB)), _tile((TN, H)), _tile((TN, H)),
                  _full((EMB, EMB)), _full((H, EMB)), _full((H, EMB)),
                  _full((1, EMB))],
        out_specs=[_tile((TN, EMB)), _full((1, EMB)), _full((1, EMB))],
        out_shape=[_sds((N, EMB)), _sds((1, EMB)), _sds((1, EMB))],
    )(h, alo, ahi, U1a, U1bl, U1bh, ub1)


def _k_node2(u1, sc, sh, U2, ub2):
    def body(u_ref, sc_ref, sh_ref, w_ref, b_ref, o_ref, s_ref, ss_ref):
        i = pl.program_id(0)
        z = jnp.maximum(u_ref[...] * sc_ref[...] + sh_ref[...], 0.0)
        o = jnp.dot(z, w_ref[...], preferred_element_type=_f32) + b_ref[...]
        o_ref[...] = o
        _acc_stats(i, o, s_ref, ss_ref)

    return pl.pallas_call(
        body,
        grid=(N // TN,),
        in_specs=[_tile((TN, EMB)), _full((1, EMB)), _full((1, EMB)),
                  _full((EMB, EMB)), _full((1, EMB))],
        out_specs=[_tile((TN, EMB)), _full((1, EMB)), _full((1, EMB))],
        out_shape=[_sds((N, EMB)), _sds((1, EMB)), _sds((1, EMB))],
    )(u1, sc, sh, U2, ub2)


def _k_node3_mid(h, u2, sc, sh, W1a, W1b):
    def body(h_ref, u_ref, sc_ref, sh_ref, wa_ref, wb_ref,
             hn_ref, p_ref, q_ref):
        hn = h_ref[...] + jnp.maximum(u_ref[...] * sc_ref[...] + sh_ref[...], 0.0)
        hn_ref[...] = hn
        p_ref[...] = jnp.dot(hn, wa_ref[...], preferred_element_type=_f32)
        q_ref[...] = jnp.dot(hn, wb_ref[...], preferred_element_type=_f32)

    return pl.pallas_call(
        body,
        grid=(N // TN,),
        in_specs=[_tile((TN, EMB)), _tile((TN, EMB)), _full((1, EMB)),
                  _full((1, EMB)), _full((EMB, EMB)), _full((EMB, EMB))],
        out_specs=[_tile((TN, EMB))] * 3,
        out_shape=[_sds((N, EMB))] * 3,
    )(h, u2, sc, sh, W1a, W1b)


def _k_node3_last(h, u2, sc, sh):
    def body(h_ref, u_ref, sc_ref, sh_ref, hn_ref):
        hn_ref[...] = h_ref[...] + jnp.maximum(
            u_ref[...] * sc_ref[...] + sh_ref[...], 0.0)

    return pl.pallas_call(
        body,
        grid=(N // TN,),
        in_specs=[_tile((TN, EMB)), _tile((TN, EMB)), _full((1, EMB)),
                  _full((1, EMB))],
        out_specs=_tile((TN, EMB)),
        out_shape=_sds((N, EMB)),
    )(h, u2, sc, sh)


def _k_pool(h, batch2):
    def body(h_ref, b_ref, o_ref):
        ids = lax.broadcasted_iota(jnp.int32, (N, G), 1)
        onehot = (b_ref[...] == ids).astype(_f32)
        dn = (((0,), (0,)), ((), ()))
        sums = lax.dot_general(onehot, h_ref[...], dn,
                               preferred_element_type=_f32)
        counts = lax.dot_general(onehot, jnp.ones((N, 1), _f32), dn,
                                 preferred_element_type=_f32)
        o_ref[...] = sums / jnp.maximum(counts, 1.0)

    return pl.pallas_call(
        body,
        in_specs=[pl.BlockSpec((N, EMB), lambda: (0, 0)),
                  pl.BlockSpec((N, 1), lambda: (0, 0))],
        out_specs=pl.BlockSpec((G, EMB), lambda: (0, 0)),
        out_shape=_sds((G, EMB)),
    )(h, batch2)


def _bn_coeffs(s, ss, count, g, be):
    mean = s / count
    var = ss / count - mean * mean
    scale = g.reshape(1, -1) * lax.rsqrt(var + EPS)
    shift = be.reshape(1, -1) - mean * scale
    return scale, shift


def kernel(x, edge_index, edge_attr, batch, Win, b_in,
           msg_W1, msg_b1, msg_g1, msg_be1, msg_W2, msg_b2, msg_g2, msg_be2,
           upd_W1, upd_b1, upd_g1, upd_be1, upd_W2, upd_b2, upd_g2, upd_be2):
    src = edge_index[0]
    dst = edge_index[1]
    batch2 = batch.reshape(N, 1)
    zeros_half = jnp.zeros((N, EMB // 2), _f32)

    W1a = [msg_W1[l][:EMB] for l in range(L)]
    W1b = [msg_W1[l][EMB:2 * EMB] for l in range(L)]
    W1c = [msg_W1[l][2 * EMB:] for l in range(L)]

    h, P, Q = _k_init(x, Win, b_in.reshape(1, -1), W1a[0], W1b[0])

    for l in range(L):
        Pg, Qg = _sc_gather(P, Q, dst, src)
        b1 = msg_b1[l].reshape(1, -1)
        s1, ss1 = _k_edge_stats(Pg, Qg, edge_attr, W1c[l], b1)
        sc1, sh1 = _bn_coeffs(s1, ss1, float(E), msg_g1[l], msg_be1[l])
        y2, s2, ss2 = _k_edge_mlp(Pg, Qg, edge_attr, W1c[l], b1, sc1, sh1,
                                  msg_W2[l], msg_b2[l].reshape(1, -1))
        sc2, sh2 = _bn_coeffs(s2, ss2, float(E), msg_g2[l], msg_be2[l])
        mlo, mhi = _k_edge_act(y2, sc2, sh2)
        alo, ahi = _sc_scatter_add(mlo, mhi, dst, zeros_half)

        u1, t1, tt1 = _k_node1(h, alo, ahi, upd_W1[l][:EMB],
                               upd_W1[l][EMB:EMB + 128],
                               upd_W1[l][EMB + 128:],
                               upd_b1[l].reshape(1, -1))
        usc1, ush1 = _bn_coeffs(t1, tt1, float(N), upd_g1[l], upd_be1[l])
        u2, t2, tt2 = _k_node2(u1, usc1, ush1, upd_W2[l],
                               upd_b2[l].reshape(1, -1))
        usc2, ush2 = _bn_coeffs(t2, tt2, float(N), upd_g2[l], upd_be2[l])
        if l < L - 1:
            h, P, Q = _k_node3_mid(h, u2, usc2, ush2, W1a[l + 1], W1b[l + 1])
        else:
            h = _k_node3_last(h, u2, usc2, ush2)

    return _k_pool(h, batch2)
